# baseline (speedup 1.0000x reference)
import functools
import math

import jax
import jax.numpy as jnp
from jax import lax
from jax.experimental import pallas as pl
from jax.experimental.pallas import tpu as pltpu
from jax.experimental.pallas import tpu_sc as plsc

D = 64
SCALE = math.sqrt(D)

NC, NS = 2, 16
NW = NC * NS
BATCH, TLEN = 4096, 200
NBB = BATCH // 128
NTT = TLEN // 8

_mesh = plsc.VectorSubcoreMesh(core_axis_name="c", subcore_axis_name="s")


@functools.partial(
    pl.kernel,
    mesh=_mesh,
    compiler_params=pltpu.CompilerParams(
        use_tc_tiling_on_sc=False, needs_layout_passes=False
    ),
    out_type=jax.ShapeDtypeStruct((TLEN, D // 8, NBB, 1024), jnp.float32),
    scratch_types=[
        pltpu.VMEM((8, 128), jnp.int32),
        pltpu.VMEM((128, D), jnp.float32),
        pltpu.VMEM((D * 128,), jnp.float32),
        pltpu.SemaphoreType.DMA,
    ],
)
def _embed(tok_hbm, table_hbm, out_hbm, tok_v, rows_v, tout_v, sem):
    j = lax.axis_index("s") * NC + lax.axis_index("c")
    lane128 = lax.iota(jnp.int32, 16) * 128

    def t_tile(tt, carry):
        pltpu.sync_copy(tok_hbm.at[tt, j], tok_v)
        for s in range(8):
            t = tt * 8 + s
            pltpu.async_copy(table_hbm.at[tok_v.at[s]], rows_v, sem).wait()

            def tok_i(i, c2):
                for g in range(D // 16):
                    vals = rows_v[i, pl.ds(g * 16, 16)] * SCALE
                    idx = lane128 + (g * 16 * 128 + i)
                    plsc.store_scatter(tout_v, [idx], vals)
                return c2

            lax.fori_loop(0, 128, tok_i, 0)
            for dt in range(D // 8):
                pltpu.sync_copy(
                    tout_v.at[pl.ds(dt * 1024, 1024)], out_hbm.at[t, dt, j]
                )
        return carry

    lax.fori_loop(0, NTT, t_tile, 0)


def kernel(tokens, table):
    tok4 = (
        tokens.astype(jnp.int32)
        .transpose(1, 0)
        .reshape(NTT, 8, NBB, 128)
        .transpose(0, 2, 1, 3)
    )
    out4 = _embed(tok4, table)
    return (
        out4.reshape(TLEN, D // 8, NBB, 8, 128)
        .transpose(2, 4, 0, 1, 3)
        .reshape(BATCH, TLEN, D)
    )


# speedup vs baseline: 1.6103x; 1.6103x over previous
import functools
import math

import jax
import jax.numpy as jnp
from jax import lax
from jax.experimental import pallas as pl
from jax.experimental.pallas import tpu as pltpu
from jax.experimental.pallas import tpu_sc as plsc

D = 64
SCALE = math.sqrt(D)

NC, NS = 2, 16
NW = NC * NS
BATCH, TLEN = 4096, 200
NBB = BATCH // 128
NTT = TLEN // 8

_mesh = plsc.VectorSubcoreMesh(core_axis_name="c", subcore_axis_name="s")


@functools.partial(
    pl.kernel,
    mesh=_mesh,
    compiler_params=pltpu.CompilerParams(
        use_tc_tiling_on_sc=False, needs_layout_passes=False
    ),
    out_type=jax.ShapeDtypeStruct((TLEN, D // 8, NBB, 1024), jnp.float32),
    scratch_types=[
        pltpu.VMEM((NTT, 8, 128), jnp.int32),
        pltpu.VMEM((2, 128, D), jnp.float32),
        pltpu.VMEM((2, 8, 1024), jnp.float32),
        pltpu.SemaphoreType.DMA,
        pltpu.SemaphoreType.DMA,
    ],
)
def _embed(tok_hbm, table_hbm, out_hbm, tok_v, rows_v, tout_v, gsem, osem):
    j = lax.axis_index("s") * NC + lax.axis_index("c")
    lane = lax.iota(jnp.int32, 16)
    lane_hi = lane // 8
    lane_lo128 = (lane % 8) * 128

    pltpu.sync_copy(tok_hbm.at[:, j], tok_v)
    pltpu.async_copy(table_hbm.at[tok_v.at[0, 0]], rows_v.at[0], gsem)

    def t_step(t, carry):
        cur = lax.rem(t, 2)
        nxt = 1 - cur
        pltpu.make_async_copy(
            table_hbm.at[tok_v.at[0, 0]], rows_v.at[cur], gsem
        ).wait()

        @pl.when(t < TLEN - 1)
        def _():
            tn = t + 1
            pltpu.async_copy(
                table_hbm.at[tok_v.at[tn // 8, tn % 8]], rows_v.at[nxt], gsem
            )

        @pl.when(t >= 2)
        def _():
            pltpu.make_async_copy(
                tout_v.at[cur], out_hbm.at[t - 2, :, j], osem
            ).wait()

        @plsc.parallel_loop(0, 128, unroll=8)
        def _(i):
            off = lane_lo128 + i
            for g in range(D // 16):
                vals = rows_v[cur, i, pl.ds(g * 16, 16)] * SCALE
                plsc.store_scatter(tout_v.at[cur], [lane_hi + 2 * g, off], vals)

        pltpu.async_copy(tout_v.at[cur], out_hbm.at[t, :, j], osem)
        return carry

    lax.fori_loop(0, TLEN, t_step, 0)
    pltpu.make_async_copy(
        tout_v.at[0], out_hbm.at[TLEN - 2, :, j], osem
    ).wait()
    pltpu.make_async_copy(
        tout_v.at[1], out_hbm.at[TLEN - 1, :, j], osem
    ).wait()


def kernel(tokens, table):
    tok4 = (
        tokens.astype(jnp.int32)
        .transpose(1, 0)
        .reshape(NTT, 8, NBB, 128)
        .transpose(0, 2, 1, 3)
    )
    out4 = _embed(tok4, table)
    return (
        out4.reshape(TLEN, D // 8, NBB, 8, 128)
        .transpose(2, 4, 0, 1, 3)
        .reshape(BATCH, TLEN, D)
    )
